# Initial kernel scaffold; baseline (speedup 1.0000x reference)
import jax
import jax.numpy as jnp
from jax.experimental import pallas as pl


def kernel(x, edge_index, gnn_ind, batch_gnn_ind, W_msg, W_self, Wq1, bq1, Wq2, bq2, Wl, bl, Wv1, bv1, Wv2, bv2):
    raise NotImplementedError("write your pallas kernel here")


# speedup vs baseline: 6.9161x; 6.9161x over previous
import functools

import jax
import jax.numpy as jnp
from jax import lax
from jax.experimental import pallas as pl
from jax.experimental.pallas import tpu as pltpu
from jax.experimental.pallas import tpu_sc as plsc

N_NODES = 10000
N_EDGES = 320000
D = 128
N_GRAPHS = 1024
BATCH = 128
N_LEMMAS = 1000

NC = 2
NS = 16
NPAD = 10240
STRIPE = NPAD // NS
E_PER_W = N_EDGES // (NC * NS)
CH = 128
N_FULL = E_PER_W // CH
TAIL = E_PER_W - N_FULL * CH


def _sc_edge_agg(x, src, dst, zrows):
    mesh = plsc.VectorSubcoreMesh(core_axis_name="c", subcore_axis_name="s")

    @functools.partial(
        pl.kernel,
        mesh=mesh,
        out_type=jax.ShapeDtypeStruct((NC, NPAD, D), jnp.float32),
        scratch_types=[
            pltpu.VMEM((CH,), jnp.int32),
            pltpu.VMEM((CH,), jnp.int32),
            pltpu.VMEM((CH, D), jnp.float32),
            pltpu.VMEM((TAIL,), jnp.int32),
            pltpu.VMEM((TAIL,), jnp.int32),
            pltpu.VMEM((TAIL, D), jnp.float32),
            pltpu.VMEM_SHARED((NPAD, D), jnp.float32),
            pltpu.SemaphoreType.DMA,
        ],
    )
    def k(x_hbm, src_hbm, dst_hbm, z_hbm, out_hbm,
          srcv, dstv, rows, srcvt, dstvt, rowst, acc, sem):
        cid = lax.axis_index("c")
        sid = lax.axis_index("s")
        pltpu.sync_copy(z_hbm, acc.at[pl.ds(sid * STRIPE, STRIPE)])
        plsc.subcore_barrier()

        edge0 = (cid * NS + sid) * E_PER_W

        def body(i, carry):
            base = edge0 + i * CH
            pltpu.sync_copy(src_hbm.at[pl.ds(base, CH)], srcv)
            pltpu.sync_copy(dst_hbm.at[pl.ds(base, CH)], dstv)
            pltpu.async_copy(x_hbm.at[srcv], rows, sem).wait()
            pltpu.sync_copy(rows, acc.at[dstv], add=True)
            return carry

        lax.fori_loop(0, N_FULL, body, 0)

        tbase = edge0 + N_FULL * CH
        pltpu.sync_copy(src_hbm.at[pl.ds(tbase, TAIL)], srcvt)
        pltpu.sync_copy(dst_hbm.at[pl.ds(tbase, TAIL)], dstvt)
        pltpu.async_copy(x_hbm.at[srcvt], rowst, sem).wait()
        pltpu.sync_copy(rowst, acc.at[dstvt], add=True)

        plsc.subcore_barrier()
        pltpu.sync_copy(acc.at[pl.ds(sid * STRIPE, STRIPE)],
                        out_hbm.at[cid, pl.ds(sid * STRIPE, STRIPE)])

    return k(x, src, dst, zrows)


NBLK = 10
BLK = N_NODES // NBLK


def _tc_body(pref, xref, gref, bgref, wmsg, wself,
             wv1, bv1, wv2, bv2, wq1, bq1, wq2, bq2, wl1, wl2, bl,
             vf_ref, log_ref, gacc):
    i = pl.program_id(0)

    @pl.when(i == 0)
    def _():
        gacc[...] = jnp.zeros_like(gacc)

    xa = pref[0] + pref[1]
    state = jnp.maximum(
        jnp.dot(xa, wmsg[...], preferred_element_type=jnp.float32)
        + jnp.dot(xref[...], wself[...], preferred_element_type=jnp.float32),
        0.0)
    g = gref[0]
    oht = (g == lax.broadcasted_iota(jnp.int32, (N_GRAPHS, BLK), 0)
           ).astype(jnp.float32)
    gacc[...] += jnp.dot(oht, state, preferred_element_type=jnp.float32)

    @pl.when(i == NBLK - 1)
    def _():
        bg = bgref[0]
        ohb = (bg == lax.broadcasted_iota(jnp.int32, (BATCH, N_GRAPHS), 0)
               ).astype(jnp.float32)
        obj = jnp.dot(ohb, gacc[...], preferred_element_type=jnp.float32)
        v = jnp.maximum(
            jnp.dot(obj, wv1[...], preferred_element_type=jnp.float32)
            + bv1[...], 0.0)
        vf_ref[...] = jax.nn.sigmoid(
            jnp.dot(v, wv2[...], preferred_element_type=jnp.float32)
            + bv2[...])
        h = jnp.dot(
            jnp.maximum(
                jnp.dot(obj, wq1[...], preferred_element_type=jnp.float32)
                + bq1[...], 0.0),
            wq2[...], preferred_element_type=jnp.float32) + bq2[...]
        q1 = jnp.maximum(obj + h[:, :D], 0.0)
        q2 = jnp.maximum(h[:, D:], 0.0)
        log_ref[...] = (
            jnp.dot(q1, wl1[...], preferred_element_type=jnp.float32)
            + jnp.dot(q2, wl2[...], preferred_element_type=jnp.float32)
            + bl[...])


def kernel(x, edge_index, gnn_ind, batch_gnn_ind, W_msg, W_self,
           Wq1, bq1, Wq2, bq2, Wl, bl, Wv1, bv1, Wv2, bv2):
    src = edge_index[0].astype(jnp.int32)
    dst = edge_index[1].astype(jnp.int32)
    zrows = jnp.zeros((STRIPE, D), jnp.float32)

    p = _sc_edge_agg(x, src, dst, zrows)

    gnn3 = gnn_ind.astype(jnp.int32).reshape(NBLK, 1, BLK)
    bgi3 = batch_gnn_ind.astype(jnp.int32).reshape(1, 1, N_GRAPHS)

    full = lambda s: pl.BlockSpec(s, lambda i: tuple(0 for _ in s))
    vf, logits = pl.pallas_call(
        _tc_body,
        grid=(NBLK,),
        in_specs=[
            pl.BlockSpec((NC, BLK, D), lambda i: (0, i, 0)),
            pl.BlockSpec((BLK, D), lambda i: (i, 0)),
            pl.BlockSpec((1, 1, BLK), lambda i: (i, 0, 0)),
            pl.BlockSpec((1, 1, N_GRAPHS), lambda i: (0, 0, 0)),
            full((D, D)), full((D, D)),
            full((D, D)), full((1, D)), full((D, 1)), full((1, 1)),
            full((D, 2 * D)), full((1, 2 * D)),
            full((2 * D, 2 * D)), full((1, 2 * D)),
            full((D, N_LEMMAS)), full((D, N_LEMMAS)), full((1, N_LEMMAS)),
        ],
        out_specs=[
            pl.BlockSpec((BATCH, 1), lambda i: (0, 0)),
            pl.BlockSpec((BATCH, N_LEMMAS), lambda i: (0, 0)),
        ],
        out_shape=[
            jax.ShapeDtypeStruct((BATCH, 1), jnp.float32),
            jax.ShapeDtypeStruct((BATCH, N_LEMMAS), jnp.float32),
        ],
        scratch_shapes=[pltpu.VMEM((N_GRAPHS, D), jnp.float32)],
    )(p, x, gnn3, bgi3, W_msg, W_self,
      Wv1[:D], bv1.reshape(1, D), Wv2, bv2.reshape(1, 1),
      Wq1[:D], bq1.reshape(1, 2 * D), Wq2, bq2.reshape(1, 2 * D),
      Wl[:D], Wl[D:], bl.reshape(1, N_LEMMAS))

    return jnp.concatenate([vf, logits], axis=1)
